# baseline (speedup 1.0000x reference)
import functools

import jax
import jax.numpy as jnp
from jax import lax
from jax.experimental import pallas as pl
from jax.experimental.pallas import tpu as pltpu
from jax.experimental.pallas import tpu_sc as plsc

_CHUNK = 128
_S_BLK = 256
_LINE = 128


def _gather_body(nc, nchunk, idx_hbm, table_hbm, out_hbm, idx_v, rows_v, sem):
    wid = lax.axis_index("s") * nc + lax.axis_index("c")
    row0 = wid * nchunk
    pltpu.sync_copy(idx_hbm.at[pl.ds(row0, nchunk)], idx_v)
    cps = [
        pltpu.async_copy(
            table_hbm.at[idx_v.at[j]],
            rows_v.at[pl.ds(j * _CHUNK, _CHUNK)],
            sem,
        )
        for j in range(nchunk)
    ]
    for cp in cps:
        cp.wait()
    pltpu.sync_copy(rows_v, out_hbm.at[pl.ds(row0 * _CHUNK, nchunk * _CHUNK)])


def _sc_gather_lines(line_index, table4):
    b = line_index.shape[0]
    info = plsc.get_sparse_core_info()
    nw = info.num_cores * info.num_subcores
    nchunk = b // (_CHUNK * nw)
    idx2d = line_index.reshape(b // _CHUNK, _CHUNK)
    k = pl.kernel(
        functools.partial(_gather_body, info.num_cores, nchunk),
        out_type=jax.ShapeDtypeStruct((b, _LINE), jnp.float32),
        mesh=plsc.VectorSubcoreMesh(core_axis_name="c", subcore_axis_name="s"),
        scratch_types=[
            pltpu.VMEM((nchunk, _CHUNK), jnp.int32),
            pltpu.VMEM((nchunk * _CHUNK, _LINE), jnp.float32),
            pltpu.SemaphoreType.DMA,
        ],
    )
    return k(idx2d, table4)


def _mulsum_body(x_ref, c4_ref, lo_ref, o_ref):
    s_blk = c4_ref.shape[0]
    c4 = c4_ref[...]
    lo = lo_ref[...]
    lane = lax.broadcasted_iota(jnp.int32, (s_blk, _LINE), 1)
    cm = jnp.where((lane // 32) == lo, c4, 0.0)
    csel = cm[:, 0:32] + cm[:, 32:64] + cm[:, 64:96] + cm[:, 96:128]
    o_ref[...] = jnp.sum(x_ref[...] * csel[:, None, :], axis=-1)


def _tc_mulsum(x, c4, lo):
    s, i, p = x.shape
    return pl.pallas_call(
        _mulsum_body,
        grid=(s // _S_BLK,),
        in_specs=[
            pl.BlockSpec((_S_BLK, i, p), lambda g: (g, 0, 0)),
            pl.BlockSpec((_S_BLK, _LINE), lambda g: (g, 0)),
            pl.BlockSpec((_S_BLK, 1), lambda g: (g, 0)),
        ],
        out_specs=pl.BlockSpec((_S_BLK, i), lambda g: (g, 0)),
        out_shape=jax.ShapeDtypeStruct((s, i), jnp.float32),
        compiler_params=pltpu.CompilerParams(
            dimension_semantics=("arbitrary",),
        ),
    )(x, c4, lo)


def kernel(x, user_index, coef):
    v, d = coef.shape
    uidx = user_index.astype(jnp.int32)
    per_line = _LINE // d
    table4 = coef.reshape(v // per_line, _LINE)
    hi = uidx // per_line
    lo = (uidx % per_line).reshape(-1, 1)
    c4 = _sc_gather_lines(hi, table4)
    return _tc_mulsum(x, c4, lo)


# speedup vs baseline: 5.0545x; 5.0545x over previous
import functools

import jax
import jax.numpy as jnp
from jax import lax
from jax.experimental import pallas as pl
from jax.experimental.pallas import tpu as pltpu
from jax.experimental.pallas import tpu_sc as plsc

_B = 16
_NBATCH = 32
_L_BLK = 1024


def _gather_body(nc, idx_hbm, tbl_hbm, out_hbm, idx_v, slab, out_v, sem):
    wid = lax.axis_index("s") * nc + lax.axis_index("c")
    row0 = wid * 4
    pltpu.sync_copy(idx_hbm.at[pl.ds(row0, 4)], idx_v)

    lane = lax.iota(jnp.int32, 16)

    for b in range(_NBATCH):
        r, c = divmod(b, 8)
        u16 = idx_v[r, pl.ds(c * _B, _B)]
        ucol = (u16 >> 7) << 7
        cps = []
        for j in range(_B):
            off = pl.multiple_of(ucol[j], 128)
            cps.append(
                pltpu.async_copy(
                    tbl_hbm.at[:, pl.ds(off, 128)],
                    slab.at[j],
                    sem,
                )
            )
        for cp in cps:
            cp.wait()
        u127 = jnp.bitwise_and(u16, 127)
        for p in range(32):
            psplat = jnp.full((16,), p, jnp.int32)
            vals = plsc.load_gather(slab, [lane, psplat, u127])
            out_v[p, pl.ds(b * _B, _B)] = vals

    pltpu.sync_copy(out_v, out_hbm.at[:, pl.ds(wid * 512, 512)])


def _sc_gather(user_index, coefT):
    b = user_index.shape[0]
    d = coefT.shape[0]
    info = plsc.get_sparse_core_info()
    nc = info.num_cores
    idx2d = user_index.reshape(128, 128)
    k = pl.kernel(
        functools.partial(_gather_body, nc),
        out_type=jax.ShapeDtypeStruct((d, b), jnp.float32),
        mesh=plsc.VectorSubcoreMesh(core_axis_name="c", subcore_axis_name="s"),
        compiler_params=pltpu.CompilerParams(needs_layout_passes=False),
        scratch_types=[
            pltpu.VMEM((4, 128), jnp.int32),
            pltpu.VMEM((_B, d, 128), jnp.float32),
            pltpu.VMEM((d, 512), jnp.float32),
            pltpu.SemaphoreType.DMA,
        ],
    )
    return k(idx2d, coefT)


def _mulsum_body(x_ref, c_ref, o_ref):
    o_ref[...] = jnp.sum(x_ref[...] * c_ref[...][None, :, :], axis=1)


def _tc_mulsum(xT, cT):
    i, p, s = xT.shape
    return pl.pallas_call(
        _mulsum_body,
        grid=(s // _L_BLK,),
        in_specs=[
            pl.BlockSpec((i, p, _L_BLK), lambda g: (0, 0, g)),
            pl.BlockSpec((p, _L_BLK), lambda g: (0, g)),
        ],
        out_specs=pl.BlockSpec((i, _L_BLK), lambda g: (0, g)),
        out_shape=jax.ShapeDtypeStruct((i, s), jnp.float32),
        compiler_params=pltpu.CompilerParams(
            dimension_semantics=("arbitrary",),
        ),
    )(xT, cT)


def kernel(x, user_index, coef):
    xT = x.transpose(1, 2, 0)
    coefT = coef.T
    uidx = user_index.astype(jnp.int32)
    cT = _sc_gather(uidx, coefT)
    outT = _tc_mulsum(xT, cT)
    return outT.T


# speedup vs baseline: 6.0007x; 1.1872x over previous
import functools

import jax
import jax.numpy as jnp
from jax import lax
from jax.experimental import pallas as pl
from jax.experimental.pallas import tpu as pltpu
from jax.experimental.pallas import tpu_sc as plsc

_B = 16
_NBATCH = 32
_SLOTS = 24
_L_BLK = 2048


def _gather_body(nc, idx_hbm, tbl_hbm, out_hbm, idx_v, slab, out_v, sem):
    wid = lax.axis_index("s") * nc + lax.axis_index("c")
    row0 = wid * 4
    pltpu.sync_copy(idx_hbm.at[pl.ds(row0, 4)], idx_v)

    lane = lax.iota(jnp.int32, 16)

    def fire(s0, n):
        cps = []
        for g0 in range(s0, s0 + n, 8):
            r, j = divmod(g0, 128)
            u16 = idx_v[r, pl.ds((j // 16) * 16, 16)]
            ucol = (u16 >> 7) << 7
            for k in range(8):
                off = pl.multiple_of(ucol[(j % 16) + k], 128)
                cps.append(
                    pltpu.async_copy(
                        tbl_hbm.at[:, pl.ds(off, 128)],
                        slab.at[(g0 + k) % _SLOTS],
                        sem,
                    )
                )
        return cps

    fifo = list(fire(0, _SLOTS))
    for eb in range(_NBATCH):
        for _ in range(_B):
            fifo.pop(0).wait()
        s0 = eb * _B
        r, c = divmod(eb, 8)
        u16 = idx_v[r, pl.ds(c * _B, _B)]
        u127 = jnp.bitwise_and(u16, 127)
        slot_vec = jnp.remainder(jnp.full((16,), s0, jnp.int32) + lane, _SLOTS)
        for p in range(32):
            psplat = jnp.full((16,), p, jnp.int32)
            vals = plsc.load_gather(slab, [slot_vec, psplat, u127])
            out_v[p, pl.ds(s0, _B)] = vals
        nxt = s0 + _SLOTS
        if nxt < 512:
            fifo.extend(fire(nxt, min(_B, 512 - nxt)))

    pltpu.sync_copy(out_v, out_hbm.at[:, pl.ds(wid * 512, 512)])


def _sc_gather(user_index, coefT):
    b = user_index.shape[0]
    d = coefT.shape[0]
    info = plsc.get_sparse_core_info()
    nc = info.num_cores
    idx2d = user_index.reshape(128, 128)
    k = pl.kernel(
        functools.partial(_gather_body, nc),
        out_type=jax.ShapeDtypeStruct((d, b), jnp.float32),
        mesh=plsc.VectorSubcoreMesh(core_axis_name="c", subcore_axis_name="s"),
        compiler_params=pltpu.CompilerParams(needs_layout_passes=False),
        scratch_types=[
            pltpu.VMEM((4, 128), jnp.int32),
            pltpu.VMEM((_SLOTS, d, 128), jnp.float32),
            pltpu.VMEM((d, 512), jnp.float32),
            pltpu.SemaphoreType.DMA,
        ],
    )
    return k(idx2d, coefT)


def _mulsum_body(x_ref, c_ref, o_ref):
    o_ref[...] = jnp.sum(x_ref[...] * c_ref[...][None, :, :], axis=1)


def _tc_mulsum(xT, cT):
    i, p, s = xT.shape
    return pl.pallas_call(
        _mulsum_body,
        grid=(s // _L_BLK,),
        in_specs=[
            pl.BlockSpec((i, p, _L_BLK), lambda g: (0, 0, g)),
            pl.BlockSpec((p, _L_BLK), lambda g: (0, g)),
        ],
        out_specs=pl.BlockSpec((i, _L_BLK), lambda g: (0, g)),
        out_shape=jax.ShapeDtypeStruct((i, s), jnp.float32),
        compiler_params=pltpu.CompilerParams(
            dimension_semantics=("arbitrary",),
        ),
    )(xT, cT)


def kernel(x, user_index, coef):
    xT = x.transpose(1, 2, 0)
    coefT = coef.T
    uidx = user_index.astype(jnp.int32)
    cT = _sc_gather(uidx, coefT)
    outT = _tc_mulsum(xT, cT)
    return outT.T
